# Initial kernel scaffold; baseline (speedup 1.0000x reference)
import jax
import jax.numpy as jnp
from jax.experimental import pallas as pl


def kernel(entity_ids, edge_index, edge_type, node_embeds, basis, comp, root, bias, a, b):
    raise NotImplementedError("write your pallas kernel here")


# speedup vs baseline: 3.1649x; 3.1649x over previous
import functools

import jax
import jax.numpy as jnp
from jax import lax
from jax.experimental import pallas as pl
from jax.experimental.pallas import tpu as pltpu
from jax.experimental.pallas import tpu_sc as plsc

N = 10000
R = 12
NBASES = 4
D = 128
E = 320000
BQ = 512
LQ = 50
PAD = 31161

NC, NS, LN = 2, 16, 16
NW = NC * NS
BATCH = 128
E_PAD = NW * BATCH * 79
EPW = E_PAD // NW
EPT = E_PAD // NS
N_PAD = 10240
SEG_PAD = 122880
ZCH = 2048
IDS_PAD = NW * BATCH * 7


def _xr_body(comp_ref, basis_ref, x_ref, out_ref):
    r = pl.program_id(0)
    w = comp_ref[r, 0] * basis_ref[0]
    for bi in range(1, NBASES):
        w = w + comp_ref[r, bi] * basis_ref[bi]
    out_ref[0] = jnp.dot(x_ref[...], w, preferred_element_type=jnp.float32)


def _xr_call(comp, basis, x):
    return pl.pallas_call(
        _xr_body,
        grid=(R, 10),
        in_specs=[
            pl.BlockSpec(memory_space=pltpu.SMEM),
            pl.BlockSpec((NBASES, D, D), lambda r, i: (0, 0, 0)),
            pl.BlockSpec((1000, D), lambda r, i: (i, 0)),
        ],
        out_specs=pl.BlockSpec((1, 1000, D), lambda r, i: (r, i, 0)),
        out_shape=jax.ShapeDtypeStruct((R, N, D), jnp.float32),
    )(comp, basis, x)


_MESH = plsc.VectorSubcoreMesh(
    core_axis_name="c", subcore_axis_name="s", num_cores=NC, num_subcores=NS)


@functools.partial(
    pl.kernel,
    out_type=jax.ShapeDtypeStruct((NC, N_PAD, D), jnp.float32),
    mesh=_MESH,
    scratch_types=(
        pltpu.VMEM_SHARED((SEG_PAD,), jnp.float32),
        pltpu.VMEM_SHARED((N_PAD, D), jnp.float32),
        pltpu.VMEM((BATCH, D), jnp.float32),
        pltpu.VMEM((ZCH,), jnp.float32),
        pltpu.VMEM((BATCH,), jnp.float32),
        pltpu.VMEM((BATCH,), jnp.int32),
        pltpu.VMEM((BATCH,), jnp.int32),
        pltpu.VMEM((1, BATCH), jnp.int32),
        pltpu.VMEM((BATCH,), jnp.int32),
        pltpu.VMEM((BATCH,), jnp.float32),
        pltpu.VMEM((BATCH,), jnp.float32),
        pltpu.SemaphoreType.DMA,
    ),
)
def _edge_agg(src_hbm, dst_hbm, rel_hbm, xr_hbm, out_hbm,
              cnt, agg, rows, zflat, ones, ia, ib, iw, ridx, fv, nrm, sem):
    c = lax.axis_index("c")
    s = lax.axis_index("s")
    w = s * NC + c

    def _zf(i, carry):
        zflat[pl.ds(i * LN, LN)] = jnp.zeros((LN,), jnp.float32)
        return carry
    lax.fori_loop(0, ZCH // LN, _zf, 0)

    for d in range(BATCH // LN):
        ones[pl.ds(d * LN, LN)] = jnp.full((LN,), 1.0, jnp.float32)

    def _zr(i, carry):
        for d in range(D // LN):
            rows[i, pl.ds(d * LN, LN)] = jnp.zeros((LN,), jnp.float32)
        return carry
    lax.fori_loop(0, BATCH, _zr, 0)

    for j in range(4):
        k = s + j * NS
        @pl.when(k < SEG_PAD // ZCH)
        def _():
            pltpu.sync_copy(zflat, cnt.at[pl.ds(k * ZCH, ZCH)])
    for j in range(5):
        k = s + j * NS
        pltpu.sync_copy(rows, agg.at[pl.ds(k * BATCH, BATCH)])

    plsc.subcore_barrier()

    def _count(j, carry):
        eb = s * EPT + j * BATCH
        pltpu.sync_copy(dst_hbm.at[pl.ds(eb, BATCH)], ia)
        pltpu.sync_copy(rel_hbm.at[pl.ds(eb, BATCH)], ib)
        for d in range(BATCH // LN):
            sl = pl.ds(d * LN, LN)
            iw[0, sl] = ia[sl] * R + ib[sl]
        pltpu.sync_copy(ones, cnt.at[iw.at[0]], add=True)
        return carry
    lax.fori_loop(0, EPT // BATCH, _count, 0)

    plsc.subcore_barrier()

    def _agg(j, carry):
        eb = w * EPW + j * BATCH
        pltpu.sync_copy(src_hbm.at[pl.ds(eb, BATCH)], ia)
        pltpu.sync_copy(rel_hbm.at[pl.ds(eb, BATCH)], ib)
        pltpu.sync_copy(dst_hbm.at[pl.ds(eb, BATCH)], iw.at[0])
        for d in range(BATCH // LN):
            sl = pl.ds(d * LN, LN)
            ridx[sl] = ib[sl] * N + ia[sl]
            ia[sl] = iw[0, sl] * R + ib[sl]
        pltpu.async_copy(cnt.at[ia], fv, sem).wait()
        for d in range(BATCH // LN):
            sl = pl.ds(d * LN, LN)
            nrm[sl] = 1.0 / jnp.maximum(fv[sl], 1.0)
        pltpu.async_copy(xr_hbm.at[ridx], rows, sem).wait()

        def _scale(g, carry2):
            nvec = nrm[pl.ds(g * LN, LN)]
            for k in range(LN):
                nk = nvec[k]
                row = g * LN + k
                for d in range(D // LN):
                    sl = pl.ds(d * LN, LN)
                    rows[row, sl] = rows[row, sl] * nk
            return carry2
        lax.fori_loop(0, BATCH // LN, _scale, 0)

        pltpu.sync_copy(rows, agg.at[iw.at[0]], add=True)
        return carry
    lax.fori_loop(0, EPW // BATCH, _agg, 0)

    plsc.subcore_barrier()

    for j in range(5):
        k = s + j * NS
        pltpu.sync_copy(agg.at[pl.ds(k * BATCH, BATCH)],
                        out_hbm.at[c, pl.ds(k * BATCH, BATCH)])


def _nodes_body(parts_ref, x_ref, root_ref, bias_ref, out_ref):
    xb = x_ref[...]
    out_ref[...] = (parts_ref[0] + parts_ref[1] + xb + bias_ref[...]
                    + jnp.dot(xb, root_ref[...], preferred_element_type=jnp.float32))


def _nodes_call(parts, x, root, bias2d):
    return pl.pallas_call(
        _nodes_body,
        grid=(10,),
        in_specs=[
            pl.BlockSpec((NC, 1000, D), lambda i: (0, i, 0)),
            pl.BlockSpec((1000, D), lambda i: (i, 0)),
            pl.BlockSpec((D, D), lambda i: (0, 0)),
            pl.BlockSpec((1, D), lambda i: (0, 0)),
        ],
        out_specs=pl.BlockSpec((1000, D), lambda i: (i, 0)),
        out_shape=jax.ShapeDtypeStruct((N, D), jnp.float32),
    )(parts, x, root, bias2d)


@functools.partial(
    pl.kernel,
    out_type=jax.ShapeDtypeStruct((IDS_PAD, D), jnp.float32),
    mesh=_MESH,
    scratch_types=(
        pltpu.VMEM((BATCH,), jnp.int32),
        pltpu.VMEM((BATCH, D), jnp.float32),
        pltpu.SemaphoreType.DMA,
    ),
)
def _entity_gather(ids_hbm, nodes_hbm, out_hbm, idxv, rows, sem):
    c = lax.axis_index("c")
    s = lax.axis_index("s")
    w = s * NC + c

    def _body(j, carry):
        base = w * (IDS_PAD // NW) + j * BATCH
        pltpu.sync_copy(ids_hbm.at[pl.ds(base, BATCH)], idxv)
        pltpu.async_copy(nodes_hbm.at[idxv], rows, sem).wait()
        pltpu.sync_copy(rows, out_hbm.at[pl.ds(base, BATCH)])
        return carry
    lax.fori_loop(0, IDS_PAD // NW // BATCH, _body, 0)


def _attn_body(h_ref, a_ref, b_ref, mb_ref, out_ref):
    av = a_ref[...]
    bv = b_ref[...]
    for k in range(8):
        h0 = h_ref[k]
        t = jnp.tanh(jnp.dot(h0, av, preferred_element_type=jnp.float32))
        e = jnp.dot(t, bv, preferred_element_type=jnp.float32) + mb_ref[k]
        m = jnp.max(e, axis=0, keepdims=True)
        p = jnp.exp(e - m)
        ws = p / jnp.sum(p, axis=0, keepdims=True)
        out_ref[k] = jnp.sum(h0 * ws, axis=0)


def _attn_call(h, a, b, mb):
    return pl.pallas_call(
        _attn_body,
        grid=(BQ // 8,),
        in_specs=[
            pl.BlockSpec((8, LQ, D), lambda i: (i, 0, 0)),
            pl.BlockSpec((D, D), lambda i: (0, 0)),
            pl.BlockSpec((D, 1), lambda i: (0, 0)),
            pl.BlockSpec((8, LQ, 1), lambda i: (i, 0, 0)),
        ],
        out_specs=pl.BlockSpec((8, D), lambda i: (i, 0)),
        out_shape=jax.ShapeDtypeStruct((BQ, D), jnp.float32),
    )(h, a, b, mb)


def kernel(entity_ids, edge_index, edge_type, node_embeds, basis, comp, root, bias, a, b):
    x = node_embeds
    xr = _xr_call(comp, basis, x).reshape(R * N, D)

    pad = E_PAD - E
    src_p = jnp.concatenate([edge_index[0].astype(jnp.int32), jnp.zeros((pad,), jnp.int32)])
    dst_p = jnp.concatenate([edge_index[1].astype(jnp.int32), jnp.full((pad,), N, jnp.int32)])
    rel_p = jnp.concatenate([edge_type.astype(jnp.int32), jnp.zeros((pad,), jnp.int32)])
    parts = _edge_agg(src_p, dst_p, rel_p, xr)

    nodes = _nodes_call(parts, x, root, bias.reshape(1, D))

    ids = entity_ids.reshape(-1).astype(jnp.int32)
    ids_p = jnp.concatenate([ids, jnp.zeros((IDS_PAD - BQ * LQ,), jnp.int32)])
    h = _entity_gather(ids_p, nodes)[:BQ * LQ].reshape(BQ, LQ, D)

    mask = (entity_ids == PAD).astype(jnp.float32)
    bm = (jnp.sum(1.0 - mask, axis=-1) > 0).astype(jnp.float32)[:, None]
    mb = (-1e30 * mask * bm).reshape(BQ, LQ, 1)
    return _attn_call(h, a, b, mb)


# speedup vs baseline: 3.4516x; 1.0906x over previous
import functools

import jax
import jax.numpy as jnp
from jax import lax
from jax.experimental import pallas as pl
from jax.experimental.pallas import tpu as pltpu
from jax.experimental.pallas import tpu_sc as plsc

N = 10000
R = 12
NBASES = 4
D = 128
E = 320000
BQ = 512
LQ = 50
PAD = 31161

NC, NS, LN = 2, 16, 16
NW = NC * NS
BATCH = 128
E_PAD = NW * BATCH * 80
EROWS = E_PAD // BATCH
EPW = E_PAD // NW
NB2 = EPW // BATCH
RPT1 = EROWS // NS
CH1 = 8
NB1 = RPT1 // CH1
N_PAD = 10112
SEG_PAD = 122880
ZCH = 2048
IDS_PAD = NW * BATCH * 7


def _xr_body(comp_ref, basis_ref, x_ref, out_ref):
    r = pl.program_id(0)
    w = comp_ref[r, 0] * basis_ref[0]
    for bi in range(1, NBASES):
        w = w + comp_ref[r, bi] * basis_ref[bi]
    out_ref[0] = jnp.dot(x_ref[...], w, preferred_element_type=jnp.float32)


def _xr_call(comp, basis, x):
    return pl.pallas_call(
        _xr_body,
        grid=(R, 10),
        in_specs=[
            pl.BlockSpec(memory_space=pltpu.SMEM),
            pl.BlockSpec((NBASES, D, D), lambda r, i: (0, 0, 0)),
            pl.BlockSpec((1000, D), lambda r, i: (i, 0)),
        ],
        out_specs=pl.BlockSpec((1, 1000, D), lambda r, i: (r, i, 0)),
        out_shape=jax.ShapeDtypeStruct((R, N, D), jnp.float32),
    )(comp, basis, x)


_MESH = plsc.VectorSubcoreMesh(
    core_axis_name="c", subcore_axis_name="s", num_cores=NC, num_subcores=NS)


@functools.partial(
    pl.kernel,
    out_type=jax.ShapeDtypeStruct((NC, N_PAD, D), jnp.float32),
    mesh=_MESH,
    scratch_types=(
        pltpu.VMEM_SHARED((SEG_PAD,), jnp.float32),
        pltpu.VMEM_SHARED((N_PAD, D), jnp.float32),
        pltpu.VMEM((BATCH, D), jnp.float32),
        pltpu.VMEM((BATCH, D), jnp.float32),
        pltpu.VMEM((ZCH,), jnp.float32),
        pltpu.VMEM((BATCH,), jnp.float32),
        pltpu.VMEM((CH1 * 3 * BATCH,), jnp.int32),
        pltpu.VMEM((CH1, BATCH), jnp.int32),
        pltpu.VMEM((3 * BATCH,), jnp.int32),
        pltpu.VMEM((3 * BATCH,), jnp.int32),
        pltpu.VMEM((1, BATCH), jnp.int32),
        pltpu.VMEM((1, BATCH), jnp.int32),
        pltpu.VMEM((BATCH,), jnp.int32),
        pltpu.VMEM((BATCH,), jnp.int32),
        pltpu.VMEM((BATCH,), jnp.int32),
        pltpu.VMEM((BATCH,), jnp.int32),
        pltpu.VMEM((BATCH,), jnp.float32),
        pltpu.VMEM((BATCH,), jnp.float32),
        pltpu.VMEM((BATCH,), jnp.float32),
        pltpu.SemaphoreType.DMA,
        pltpu.SemaphoreType.DMA,
        pltpu.SemaphoreType.DMA,
        pltpu.SemaphoreType.DMA,
        pltpu.SemaphoreType.DMA,
        pltpu.SemaphoreType.DMA,
        pltpu.SemaphoreType.DMA,
    ),
)
def _edge_agg(packed_hbm, xr_hbm, out_hbm,
              cnt, agg, rows0, rows1, zflat, ones, pb1, iwb, pk0, pk1,
              iw0, iw1, seg0, seg1, ridx0, ridx1, fv0, fv1, nrm,
              lsem, csem0, csem1, gsem0, gsem1, ssem0, ssem1):
    c = lax.axis_index("c")
    s = lax.axis_index("s")
    w = s * NC + c

    def _zf(i, carry):
        zflat[pl.ds(i * LN, LN)] = jnp.zeros((LN,), jnp.float32)
        return carry
    lax.fori_loop(0, ZCH // LN, _zf, 0)

    for d in range(BATCH // LN):
        ones[pl.ds(d * LN, LN)] = jnp.full((LN,), 1.0, jnp.float32)

    def _zr(i, carry):
        for d in range(D // LN):
            rows0[i, pl.ds(d * LN, LN)] = jnp.zeros((LN,), jnp.float32)
        return carry
    lax.fori_loop(0, BATCH, _zr, 0)

    for j in range(4):
        k = s + j * NS
        @pl.when(k < SEG_PAD // ZCH)
        def _():
            pltpu.sync_copy(zflat, cnt.at[pl.ds(k * ZCH, ZCH)])
    for j in range(5):
        k = s + j * NS
        @pl.when(k < N_PAD // BATCH)
        def _():
            pltpu.sync_copy(rows0, agg.at[pl.ds(k * BATCH, BATCH)])

    plsc.subcore_barrier()

    def _count(i, carry):
        rb = s * RPT1 + i * CH1
        pltpu.sync_copy(packed_hbm.at[pl.ds(rb * 3 * BATCH, CH1 * 3 * BATCH)], pb1)

        def _cseg(r, carry2):
            base = r * 3 * BATCH
            for d in range(BATCH // LN):
                sl = pl.ds(d * LN, LN)
                iwb[r, sl] = (pb1[pl.ds(base + BATCH + d * LN, LN)] * R
                              + pb1[pl.ds(base + 2 * BATCH + d * LN, LN)])
            return carry2
        lax.fori_loop(0, CH1, _cseg, 0)

        for r in range(CH1):
            pltpu.async_copy(ones, cnt.at[iwb.at[r]], lsem, add=True)
        for r in range(CH1):
            pltpu.make_async_copy(ones, cnt.at[iwb.at[0]], lsem).wait()
        return carry
    lax.fori_loop(0, NB1, _count, 0)

    plsc.subcore_barrier()

    def _prep(j, pk, iw, seg, ridx, fv, csem):
        pltpu.sync_copy(packed_hbm.at[pl.ds((w * NB2 + j) * 3 * BATCH, 3 * BATCH)], pk)
        for d in range(BATCH // LN):
            sl = pl.ds(d * LN, LN)
            srcv = pk[pl.ds(d * LN, LN)]
            dstv = pk[pl.ds(BATCH + d * LN, LN)]
            relv = pk[pl.ds(2 * BATCH + d * LN, LN)]
            ridx[sl] = relv * N + srcv
            seg[sl] = dstv * R + relv
            iw[0, sl] = dstv
        pltpu.async_copy(cnt.at[seg], fv, csem)

    def _start_gather(ridx, rows, gsem):
        pltpu.async_copy(xr_hbm.at[ridx], rows, gsem)

    def _process(rows, iw, seg, ridx, fv, csem, gsem, ssem):
        pltpu.make_async_copy(cnt.at[seg], fv, csem).wait()
        pltpu.make_async_copy(xr_hbm.at[ridx], rows, gsem).wait()
        for d in range(BATCH // LN):
            sl = pl.ds(d * LN, LN)
            nrm[sl] = 1.0 / jnp.maximum(fv[sl], 1.0)

        def _scale(g, carry2):
            nvec = nrm[pl.ds(g * LN, LN)]
            for k in range(LN):
                nk = nvec[k]
                row = g * LN + k
                for d in range(D // LN):
                    sl = pl.ds(d * LN, LN)
                    rows[row, sl] = rows[row, sl] * nk
            return carry2
        lax.fori_loop(0, BATCH // LN, _scale, 0)

        pltpu.async_copy(rows, agg.at[iw.at[0]], ssem, add=True)

    def _wait_scatter(rows, iw, ssem):
        pltpu.make_async_copy(rows, agg.at[iw.at[0]], ssem).wait()

    _prep(0, pk0, iw0, seg0, ridx0, fv0, csem0)
    _start_gather(ridx0, rows0, gsem0)

    def _pipe(t, carry):
        _process(rows0, iw0, seg0, ridx0, fv0, csem0, gsem0, ssem0)

        @pl.when(t > 0)
        def _():
            _wait_scatter(rows1, iw1, ssem1)
        _prep(2 * t + 1, pk1, iw1, seg1, ridx1, fv1, csem1)
        _start_gather(ridx1, rows1, gsem1)

        _process(rows1, iw1, seg1, ridx1, fv1, csem1, gsem1, ssem1)

        @pl.when(t < NB2 // 2 - 1)
        def _():
            _wait_scatter(rows0, iw0, ssem0)
            _prep(2 * t + 2, pk0, iw0, seg0, ridx0, fv0, csem0)
            _start_gather(ridx0, rows0, gsem0)
        return carry
    lax.fori_loop(0, NB2 // 2, _pipe, 0)

    _wait_scatter(rows0, iw0, ssem0)
    _wait_scatter(rows1, iw1, ssem1)

    plsc.subcore_barrier()

    for j in range(5):
        k = s + j * NS
        @pl.when(k < N_PAD // BATCH)
        def _():
            pltpu.sync_copy(agg.at[pl.ds(k * BATCH, BATCH)],
                            out_hbm.at[c, pl.ds(k * BATCH, BATCH)])


def _nodes_body(parts_ref, x_ref, root_ref, bias_ref, out_ref):
    xb = x_ref[...]
    out_ref[...] = (parts_ref[0] + parts_ref[1] + xb + bias_ref[...]
                    + jnp.dot(xb, root_ref[...], preferred_element_type=jnp.float32))


def _nodes_call(parts, x, root, bias2d):
    return pl.pallas_call(
        _nodes_body,
        grid=(10,),
        in_specs=[
            pl.BlockSpec((NC, 1000, D), lambda i: (0, i, 0)),
            pl.BlockSpec((1000, D), lambda i: (i, 0)),
            pl.BlockSpec((D, D), lambda i: (0, 0)),
            pl.BlockSpec((1, D), lambda i: (0, 0)),
        ],
        out_specs=pl.BlockSpec((1000, D), lambda i: (i, 0)),
        out_shape=jax.ShapeDtypeStruct((N, D), jnp.float32),
    )(parts, x, root, bias2d)


_GB = 7


@functools.partial(
    pl.kernel,
    out_type=jax.ShapeDtypeStruct((IDS_PAD, D), jnp.float32),
    mesh=_MESH,
    scratch_types=(
        pltpu.VMEM((_GB * BATCH,), jnp.int32),
        pltpu.VMEM((2, BATCH, D), jnp.float32),
        pltpu.SemaphoreType.DMA,
        pltpu.SemaphoreType.DMA,
        pltpu.SemaphoreType.DMA,
        pltpu.SemaphoreType.DMA,
    ),
)
def _entity_gather(ids_hbm, nodes_hbm, out_hbm, idx2, rb, gs0, gs1, os0, os1):
    c = lax.axis_index("c")
    s = lax.axis_index("s")
    w = s * NC + c
    gs = (gs0, gs1)
    os_ = (os0, os1)

    pltpu.sync_copy(ids_hbm.at[pl.ds(w * _GB * BATCH, _GB * BATCH)], idx2)
    pltpu.async_copy(nodes_hbm.at[idx2.at[pl.ds(0, BATCH)]], rb.at[0], gs[0])
    for j in range(_GB):
        p = j & 1
        pltpu.make_async_copy(
            nodes_hbm.at[idx2.at[pl.ds(j * BATCH, BATCH)]], rb.at[p], gs[p]).wait()
        pltpu.async_copy(rb.at[p], out_hbm.at[pl.ds((w * _GB + j) * BATCH, BATCH)], os_[p])
        if j + 1 < _GB:
            if j >= 1:
                pltpu.make_async_copy(
                    rb.at[1 - p],
                    out_hbm.at[pl.ds((w * _GB + j - 1) * BATCH, BATCH)], os_[1 - p]).wait()
            pltpu.async_copy(
                nodes_hbm.at[idx2.at[pl.ds((j + 1) * BATCH, BATCH)]], rb.at[1 - p], gs[1 - p])
    pltpu.make_async_copy(rb.at[1], out_hbm.at[pl.ds(0, BATCH)], os_[1]).wait()
    pltpu.make_async_copy(rb.at[0], out_hbm.at[pl.ds(0, BATCH)], os_[0]).wait()


def _attn_body(h_ref, a_ref, b_ref, mb_ref, out_ref):
    av = a_ref[...]
    bv = b_ref[...]
    for k in range(8):
        h0 = h_ref[k]
        t = jnp.tanh(jnp.dot(h0, av, preferred_element_type=jnp.float32))
        e = jnp.dot(t, bv, preferred_element_type=jnp.float32) + mb_ref[k]
        m = jnp.max(e, axis=0, keepdims=True)
        p = jnp.exp(e - m)
        ws = p / jnp.sum(p, axis=0, keepdims=True)
        out_ref[k] = jnp.sum(h0 * ws, axis=0)


def _attn_call(h, a, b, mb):
    return pl.pallas_call(
        _attn_body,
        grid=(BQ // 8,),
        in_specs=[
            pl.BlockSpec((8, LQ, D), lambda i: (i, 0, 0)),
            pl.BlockSpec((D, D), lambda i: (0, 0)),
            pl.BlockSpec((D, 1), lambda i: (0, 0)),
            pl.BlockSpec((8, LQ, 1), lambda i: (i, 0, 0)),
        ],
        out_specs=pl.BlockSpec((8, D), lambda i: (i, 0)),
        out_shape=jax.ShapeDtypeStruct((BQ, D), jnp.float32),
    )(h, a, b, mb)


def kernel(entity_ids, edge_index, edge_type, node_embeds, basis, comp, root, bias, a, b):
    x = node_embeds
    xr = _xr_call(comp, basis, x).reshape(R * N, D)

    pad = E_PAD - E
    src_p = jnp.concatenate([edge_index[0].astype(jnp.int32), jnp.zeros((pad,), jnp.int32)])
    dst_p = jnp.concatenate([edge_index[1].astype(jnp.int32), jnp.full((pad,), N, jnp.int32)])
    rel_p = jnp.concatenate([edge_type.astype(jnp.int32), jnp.zeros((pad,), jnp.int32)])
    packed = (jnp.stack([src_p, dst_p, rel_p], axis=0)
              .reshape(3, EROWS, BATCH).transpose(1, 0, 2).reshape(-1))
    parts = _edge_agg(packed, xr)

    nodes = _nodes_call(parts, x, root, bias.reshape(1, D))

    ids = entity_ids.reshape(-1).astype(jnp.int32)
    ids_p = jnp.concatenate([ids, jnp.zeros((IDS_PAD - BQ * LQ,), jnp.int32)])
    h = _entity_gather(ids_p, nodes)[:BQ * LQ].reshape(BQ, LQ, D)

    mask = (entity_ids == PAD).astype(jnp.float32)
    bm = (jnp.sum(1.0 - mask, axis=-1) > 0).astype(jnp.float32)[:, None]
    mb = (-1e30 * mask * bm).reshape(BQ, LQ, 1)
    return _attn_call(h, a, b, mb)
